# Initial kernel scaffold; baseline (speedup 1.0000x reference)
import functools

import jax
import jax.numpy as jnp
from jax import lax
from jax.experimental import pallas as pl
from jax.experimental.pallas import tpu as pltpu
from jax.experimental.pallas import tpu_sc as plsc

N = 10000
E = 320000
F_IN = 128
HID = 128
NCLASS = 16

NC = 2
NS = 16
NW = NC * NS
EPW = E // NW
K = 80
NCH = EPW // K
RPS = N // NS


def _mm_body(x_ref, w_ref, o_ref):
    o_ref[...] = jnp.dot(x_ref[...], w_ref[...],
                         preferred_element_type=jnp.float32)


def _matmul(x, w):
    bm = 512
    grid = (N + bm - 1) // bm
    return pl.pallas_call(
        _mm_body,
        grid=(grid,),
        in_specs=[
            pl.BlockSpec((bm, F_IN), lambda i: (i, 0)),
            pl.BlockSpec((F_IN, HID), lambda i: (0, 0)),
        ],
        out_specs=pl.BlockSpec((bm, HID), lambda i: (i, 0)),
        out_shape=jax.ShapeDtypeStruct((N, HID), jnp.float32),
    )(x, w)


def _edge_body(h_hbm, src_hbm, dst_hbm, w_hbm, accp_hbm, cwp_hbm,
               srcb, dstb, wb, rows, wrow, acc_sh, cw_sh, sem):
    cid = lax.axis_index("c")
    sid = lax.axis_index("s")
    wid = cid * NS + sid

    zv = jnp.zeros((16,), jnp.float32)
    for g in range(K):
        wrow[g, :] = zv
        for k in range(HID // 16):
            rows[g, pl.ds(k * 16, 16)] = zv
    for t in range(8):
        base = sid * RPS + t * K
        sz = K if t < 7 else RPS - 7 * K
        pltpu.sync_copy(rows.at[pl.ds(0, sz)], acc_sh.at[pl.ds(base, sz)])
        pltpu.sync_copy(wrow.at[pl.ds(0, sz)], cw_sh.at[pl.ds(base, sz)])
    plsc.subcore_barrier()

    pltpu.sync_copy(src_hbm.at[pl.ds(wid * NCH, NCH)], srcb)
    pltpu.sync_copy(dst_hbm.at[pl.ds(wid * NCH, NCH)], dstb)
    pltpu.sync_copy(w_hbm.at[pl.ds(wid * NCH, NCH)], wb)

    def chunk(j, carry):
        pltpu.async_copy(h_hbm.at[srcb.at[j]], rows, sem).wait()
        jv = jnp.full((16,), j, jnp.int32)
        for g in range(K):
            wv = plsc.load_gather(wb, [jv, jnp.full((16,), g, jnp.int32)])
            wrow[g, :] = wv
            for k in range(HID // 16):
                rows[g, pl.ds(k * 16, 16)] = rows[g, pl.ds(k * 16, 16)] * wv
        pltpu.sync_copy(rows, acc_sh.at[dstb.at[j]], add=True)
        pltpu.sync_copy(wrow, cw_sh.at[srcb.at[j]], add=True)
        return carry

    lax.fori_loop(0, NCH, chunk, 0)
    plsc.subcore_barrier()

    pltpu.sync_copy(acc_sh.at[pl.ds(sid * RPS, RPS)],
                    accp_hbm.at[cid, pl.ds(sid * RPS, RPS)])
    pltpu.sync_copy(cw_sh.at[pl.ds(sid * RPS, RPS)],
                    cwp_hbm.at[cid, pl.ds(sid * RPS, RPS)])


_edge_kernel = functools.partial(
    pl.kernel,
    out_type=[
        jax.ShapeDtypeStruct((NC, N, HID), jnp.float32),
        jax.ShapeDtypeStruct((NC, N, 16), jnp.float32),
    ],
    mesh=plsc.VectorSubcoreMesh(core_axis_name="c", subcore_axis_name="s"),
    scratch_types=[
        pltpu.VMEM((NCH, K), jnp.int32),
        pltpu.VMEM((NCH, K), jnp.int32),
        pltpu.VMEM((NCH, K), jnp.float32),
        pltpu.VMEM((K, HID), jnp.float32),
        pltpu.VMEM((K, 16), jnp.float32),
        pltpu.VMEM_SHARED((N, HID), jnp.float32),
        pltpu.VMEM_SHARED((N, 16), jnp.float32),
        pltpu.SemaphoreType.DMA,
    ],
)(_edge_body)


def _finish_body(p_ref, cw_ref, w2_ref, o_ref):
    acc = p_ref[0] + p_ref[1]
    r = jnp.maximum(acc, 0.0)
    c = cw_ref[0, :, :1] + cw_ref[1, :, :1]
    s = jnp.sum(r * c, axis=0, keepdims=True)
    o_ref[...] = jnp.dot(s, w2_ref[...],
                         preferred_element_type=jnp.float32) * (1.0 / N)


def _finish(p, cw, w2):
    return pl.pallas_call(
        _finish_body,
        out_shape=jax.ShapeDtypeStruct((1, NCLASS), jnp.float32),
    )(p, cw, w2)


def kernel(x, edge_index, edge_weight, W1, W2):
    h = _matmul(x, W1)
    src2d = edge_index[0].reshape(E // K, K)
    dst2d = edge_index[1].reshape(E // K, K)
    w2d = edge_weight.reshape(E // K, K)
    accp, cwp = _edge_kernel(h, src2d, dst2d, w2d)
    return _finish(accp, cwp, W2)


# speedup vs baseline: 7.2345x; 7.2345x over previous
import functools

import jax
import jax.numpy as jnp
from jax import lax
from jax.experimental import pallas as pl
from jax.experimental.pallas import tpu as pltpu
from jax.experimental.pallas import tpu_sc as plsc

N = 10000
E = 320000
F_IN = 128
HID = 128
FH = HID // 2
NCLASS = 16

NC = 2
NS = 16
EPS = E // NS
K = 80
NCH = EPS // K
RPS = N // NS


def _mm_body(x_ref, w_ref, o_ref):
    o_ref[0] = jnp.dot(x_ref[...], w_ref[0],
                       preferred_element_type=jnp.float32)


def _matmul_split(x, w):
    bm = 400
    grid = (N // bm, NC)
    return pl.pallas_call(
        _mm_body,
        grid=grid,
        in_specs=[
            pl.BlockSpec((bm, F_IN), lambda i, c: (i, 0)),
            pl.BlockSpec((1, F_IN, FH), lambda i, c: (c, 0, 0)),
        ],
        out_specs=pl.BlockSpec((1, bm, FH), lambda i, c: (c, i, 0)),
        out_shape=jax.ShapeDtypeStruct((NC, N, FH), jnp.float32),
    )(x, w)


def _edge_body(h_hbm, src_hbm, dst_hbm, w_hbm, accp_hbm, cwp_hbm,
               srcb, dstb, wb, rows, wrow, acc_sh, cw_sh, sem):
    cid = lax.axis_index("c")
    sid = lax.axis_index("s")

    zv = jnp.zeros((16,), jnp.float32)
    for g in range(K):
        wrow[g, :] = zv
        for k in range(FH // 16):
            rows[g, pl.ds(k * 16, 16)] = zv
    for t in range(8):
        base = sid * RPS + t * K
        sz = K if t < 7 else RPS - 7 * K
        pltpu.sync_copy(rows.at[pl.ds(0, sz)], acc_sh.at[pl.ds(base, sz)])

        @pl.when(cid == 0)
        def _():
            pltpu.sync_copy(wrow.at[pl.ds(0, sz)], cw_sh.at[pl.ds(base, sz)])

    plsc.subcore_barrier()

    pltpu.sync_copy(src_hbm.at[sid], srcb)
    pltpu.sync_copy(dst_hbm.at[sid], dstb)
    pltpu.sync_copy(w_hbm.at[sid], wb)

    def chunk(j, carry):
        pltpu.async_copy(h_hbm.at[cid].at[srcb.at[j]], rows, sem).wait()
        for g16 in range(K // 16):
            wv16 = wb[j, pl.ds(g16 * 16, 16)]
            for l in range(16):
                g = g16 * 16 + l
                wv = jnp.full((16,), wv16[l])
                wrow[g, :] = wv
                for k in range(FH // 16):
                    rows[g, pl.ds(k * 16, 16)] = rows[g, pl.ds(k * 16, 16)] * wv
        pltpu.sync_copy(rows, acc_sh.at[dstb.at[j]], add=True)

        @pl.when(cid == 0)
        def _():
            pltpu.sync_copy(wrow, cw_sh.at[srcb.at[j]], add=True)

        return carry

    lax.fori_loop(0, NCH, chunk, 0)
    plsc.subcore_barrier()

    pltpu.sync_copy(acc_sh.at[pl.ds(sid * RPS, RPS)], accp_hbm.at[cid, sid])

    @pl.when(cid == 0)
    def _():
        pltpu.sync_copy(cw_sh.at[pl.ds(sid * RPS, RPS)], cwp_hbm.at[sid])


_edge_kernel = functools.partial(
    pl.kernel,
    out_type=[
        jax.ShapeDtypeStruct((NC, NS, RPS, FH), jnp.float32),
        jax.ShapeDtypeStruct((NS, RPS, 16), jnp.float32),
    ],
    mesh=plsc.VectorSubcoreMesh(core_axis_name="c", subcore_axis_name="s"),
    compiler_params=pltpu.CompilerParams(use_tc_tiling_on_sc=False),
    scratch_types=[
        pltpu.VMEM((NCH, K), jnp.int32),
        pltpu.VMEM((NCH, K), jnp.int32),
        pltpu.VMEM((NCH, K), jnp.float32),
        pltpu.VMEM((K, FH), jnp.float32),
        pltpu.VMEM((K, 16), jnp.float32),
        pltpu.VMEM_SHARED((N, FH), jnp.float32),
        pltpu.VMEM_SHARED((N, 16), jnp.float32),
        pltpu.SemaphoreType.DMA,
    ],
)(_edge_body)


def _finish_body(p_ref, cw_ref, w2_ref, o_ref):
    c = cw_ref[:, :1]
    r0 = jnp.maximum(p_ref[0], 0.0) * c
    r1 = jnp.maximum(p_ref[1], 0.0) * c
    s = jnp.concatenate(
        [jnp.sum(r0, axis=0, keepdims=True),
         jnp.sum(r1, axis=0, keepdims=True)], axis=1)
    o_ref[...] = jnp.dot(s, w2_ref[...],
                         preferred_element_type=jnp.float32) * (1.0 / N)


def _finish(p, cw, w2):
    return pl.pallas_call(
        _finish_body,
        out_shape=jax.ShapeDtypeStruct((1, NCLASS), jnp.float32),
    )(p, cw, w2)


def kernel(x, edge_index, edge_weight, W1, W2):
    w1s = W1.reshape(F_IN, NC, FH).transpose(1, 0, 2)
    h = _matmul_split(x, w1s)
    src3d = edge_index[0].reshape(NS, NCH, K)
    dst3d = edge_index[1].reshape(NS, NCH, K)
    w3d = edge_weight.reshape(NS, NCH, K)
    accp, cwp = _edge_kernel(h, src3d, dst3d, w3d)
    accp = accp.reshape(NC, N, FH)
    cwp = cwp.reshape(N, 16)
    return _finish(accp, cwp, W2)


# speedup vs baseline: 13.7785x; 1.9046x over previous
import functools

import jax
import jax.numpy as jnp
from jax import lax
from jax.experimental import pallas as pl
from jax.experimental.pallas import tpu as pltpu
from jax.experimental.pallas import tpu_sc as plsc

N = 10000
E = 320000
F_IN = 128
HID = 128
FH = HID // 2
NCLASS = 16

NC = 2
NS = 16
EPS = E // NS
K = 80
NCH = EPS // K
RPS = N // NS
NPAD = 10240


def _mm_body(x_ref, w_ref, o_ref):
    o_ref[0] = jnp.dot(x_ref[...], w_ref[0],
                       preferred_element_type=jnp.float32)


def _matmul_split(x, w):
    bm = 400
    grid = (N // bm, NC)
    return pl.pallas_call(
        _mm_body,
        grid=grid,
        in_specs=[
            pl.BlockSpec((bm, F_IN), lambda i, c: (i, 0)),
            pl.BlockSpec((1, F_IN, FH), lambda i, c: (c, 0, 0)),
        ],
        out_specs=pl.BlockSpec((1, bm, FH), lambda i, c: (c, i, 0)),
        out_shape=jax.ShapeDtypeStruct((NC, N, FH), jnp.float32),
    )(x, w)


def _edge_body(h_hbm, src_hbm, dst_hbm, w_hbm, accp_hbm, cp_hbm,
               srcb, dstb, wb, gbuf0, gbuf1, mbuf0, mbuf1, c_loc,
               acc_sh, gsem0, gsem1, ssem0, ssem1):
    cid = lax.axis_index("c")
    sid = lax.axis_index("s")
    gbuf = (gbuf0, gbuf1)
    mbuf = (mbuf0, mbuf1)
    gsem = (gsem0, gsem1)
    ssem = (ssem0, ssem1)

    zv = jnp.zeros((16,), jnp.float32)
    for p in range(2):
        for g in range(K):
            for k in range(FH // 16):
                mbuf[p][g, pl.ds(k * 16, 16)] = zv

    def zc(i, carry):
        c_loc[pl.ds(i * 16, 16)] = zv
        return carry

    lax.fori_loop(0, NPAD // 16, zc, 0)

    for t in range(8):
        base = sid * RPS + t * K
        sz = K if t < 7 else RPS - 7 * K
        pltpu.sync_copy(mbuf0.at[pl.ds(0, sz)], acc_sh.at[pl.ds(base, sz)])
    plsc.subcore_barrier()

    pltpu.sync_copy(src_hbm.at[sid], srcb)
    pltpu.sync_copy(dst_hbm.at[sid], dstb)
    pltpu.sync_copy(w_hbm.at[sid], wb)

    pltpu.async_copy(h_hbm.at[cid].at[srcb.at[0]], gbuf0, gsem0)
    for p in range(2):
        pltpu.async_copy(mbuf[p], acc_sh.at[dstb.at[p]], ssem[p], add=True)

    def step(t, carry):
        for p in range(2):
            j = 2 * t + p
            @pl.when(j + 1 < NCH)
            def _():
                pltpu.async_copy(h_hbm.at[cid].at[srcb.at[j + 1]],
                                 gbuf[1 - p], gsem[1 - p])

            pltpu.make_async_copy(h_hbm.at[cid].at[srcb.at[j]],
                                  gbuf[p], gsem[p]).wait()
            pltpu.make_async_copy(mbuf[p], acc_sh.at[dstb.at[j]],
                                  ssem[p]).wait()

            for g16 in range(K // 16):
                wv16 = wb[j, pl.ds(g16 * 16, 16)]
                idxv = srcb[j, pl.ds(g16 * 16, 16)]
                plsc.addupdate_scatter(c_loc, [idxv], wv16)
                for l in range(16):
                    g = g16 * 16 + l
                    wv = jnp.full((16,), wv16[l])
                    for k in range(FH // 16):
                        mbuf[p][g, pl.ds(k * 16, 16)] = (
                            gbuf[p][g, pl.ds(k * 16, 16)] * wv)
            pltpu.async_copy(mbuf[p], acc_sh.at[dstb.at[j]], ssem[p],
                             add=True)
        return carry

    lax.fori_loop(0, NCH // 2, step, 0)

    for p in range(2):
        pltpu.make_async_copy(mbuf[p], acc_sh.at[dstb.at[NCH - 2 + p]],
                              ssem[p]).wait()
    plsc.subcore_barrier()

    pltpu.sync_copy(acc_sh.at[pl.ds(sid * RPS, RPS)], accp_hbm.at[cid, sid])

    @pl.when(cid == 0)
    def _():
        pltpu.sync_copy(c_loc, cp_hbm.at[sid])


_edge_kernel = functools.partial(
    pl.kernel,
    out_type=[
        jax.ShapeDtypeStruct((NC, NS, RPS, FH), jnp.float32),
        jax.ShapeDtypeStruct((NS, NPAD), jnp.float32),
    ],
    mesh=plsc.VectorSubcoreMesh(core_axis_name="c", subcore_axis_name="s"),
    compiler_params=pltpu.CompilerParams(use_tc_tiling_on_sc=False,
                                         needs_layout_passes=False),
    scratch_types=[
        pltpu.VMEM((NCH, K), jnp.int32),
        pltpu.VMEM((NCH, K), jnp.int32),
        pltpu.VMEM((NCH, K), jnp.float32),
        pltpu.VMEM((K, FH), jnp.float32),
        pltpu.VMEM((K, FH), jnp.float32),
        pltpu.VMEM((K, FH), jnp.float32),
        pltpu.VMEM((K, FH), jnp.float32),
        pltpu.VMEM((NPAD,), jnp.float32),
        pltpu.VMEM_SHARED((N, FH), jnp.float32),
        pltpu.SemaphoreType.DMA,
        pltpu.SemaphoreType.DMA,
        pltpu.SemaphoreType.DMA,
        pltpu.SemaphoreType.DMA,
    ],
)(_edge_body)


def _finish_body(p_ref, cp_ref, w2_ref, o_ref):
    c = jnp.sum(cp_ref[...], axis=0)[:N, None]
    r0 = jnp.maximum(p_ref[0], 0.0) * c
    r1 = jnp.maximum(p_ref[1], 0.0) * c
    s = jnp.concatenate(
        [jnp.sum(r0, axis=0, keepdims=True),
         jnp.sum(r1, axis=0, keepdims=True)], axis=1)
    o_ref[...] = jnp.dot(s, w2_ref[...],
                         preferred_element_type=jnp.float32) * (1.0 / N)


def _finish(p, cp, w2):
    return pl.pallas_call(
        _finish_body,
        out_shape=jax.ShapeDtypeStruct((1, NCLASS), jnp.float32),
    )(p, cp, w2)


def kernel(x, edge_index, edge_weight, W1, W2):
    w1s = W1.reshape(F_IN, NC, FH).transpose(1, 0, 2)
    h = _matmul_split(x, w1s)
    src3d = edge_index[0].reshape(NS, NCH, K)
    dst3d = edge_index[1].reshape(NS, NCH, K)
    w3d = edge_weight.reshape(NS, NCH, K)
    accp, cp = _edge_kernel(h, src3d, dst3d, w3d)
    accp = accp.reshape(NC, N, FH)
    return _finish(accp, cp, W2)


# speedup vs baseline: 15.1955x; 1.1028x over previous
import functools

import jax
import jax.numpy as jnp
from jax import lax
from jax.experimental import pallas as pl
from jax.experimental.pallas import tpu as pltpu
from jax.experimental.pallas import tpu_sc as plsc

N = 10000
E = 320000
F_IN = 128
HID = 128
FH = HID // 2
NCLASS = 16

NC = 2
NS = 16
EPS = E // NS
K = 80
NCH = EPS // K
RPS = N // NS
NPAD = 10240


def _edge_body(h_hbm, src_hbm, dst_hbm, w_hbm, accp_hbm, cp_hbm,
               srcb, dstb, wb, gbuf0, gbuf1, mbuf0, mbuf1, c_loc,
               acc_sh, gsem0, gsem1, ssem0, ssem1):
    cid = lax.axis_index("c")
    sid = lax.axis_index("s")
    gbuf = (gbuf0, gbuf1)
    mbuf = (mbuf0, mbuf1)
    gsem = (gsem0, gsem1)
    ssem = (ssem0, ssem1)

    zv = jnp.zeros((16,), jnp.float32)
    for p in range(2):
        for g in range(K):
            for k in range(FH // 16):
                mbuf[p][g, pl.ds(k * 16, 16)] = zv

    def zc(i, carry):
        c_loc[pl.ds(i * 16, 16)] = zv
        return carry

    lax.fori_loop(0, NPAD // 16, zc, 0)

    for t in range(8):
        base = sid * RPS + t * K
        sz = K if t < 7 else RPS - 7 * K
        pltpu.sync_copy(mbuf0.at[pl.ds(0, sz)], acc_sh.at[pl.ds(base, sz)])
    plsc.subcore_barrier()

    pltpu.sync_copy(src_hbm.at[sid], srcb)
    pltpu.sync_copy(dst_hbm.at[sid], dstb)
    pltpu.sync_copy(w_hbm.at[sid], wb)

    pltpu.async_copy(h_hbm.at[cid].at[srcb.at[0]], gbuf0, gsem0)
    for p in range(2):
        pltpu.async_copy(mbuf[p], acc_sh.at[dstb.at[p]], ssem[p], add=True)

    def step(t, carry):
        for p in range(2):
            j = 2 * t + p
            @pl.when(j + 1 < NCH)
            def _():
                pltpu.async_copy(h_hbm.at[cid].at[srcb.at[j + 1]],
                                 gbuf[1 - p], gsem[1 - p])

            pltpu.make_async_copy(h_hbm.at[cid].at[srcb.at[j]],
                                  gbuf[p], gsem[p]).wait()
            pltpu.make_async_copy(mbuf[p], acc_sh.at[dstb.at[j]],
                                  ssem[p]).wait()

            for g16 in range(K // 16):
                wv16 = wb[j, pl.ds(g16 * 16, 16)]
                idxv = srcb[j, pl.ds(g16 * 16, 16)]
                plsc.addupdate_scatter(c_loc, [idxv], wv16)
                for l in range(16):
                    g = g16 * 16 + l
                    wv = jnp.full((16,), wv16[l])
                    for k in range(FH // 16):
                        mbuf[p][g, pl.ds(k * 16, 16)] = (
                            gbuf[p][g, pl.ds(k * 16, 16)] * wv)
            pltpu.async_copy(mbuf[p], acc_sh.at[dstb.at[j]], ssem[p],
                             add=True)
        return carry

    lax.fori_loop(0, NCH // 2, step, 0)

    for p in range(2):
        pltpu.make_async_copy(mbuf[p], acc_sh.at[dstb.at[NCH - 2 + p]],
                              ssem[p]).wait()
    plsc.subcore_barrier()

    pltpu.sync_copy(acc_sh.at[pl.ds(sid * RPS, RPS)], accp_hbm.at[cid, sid])

    @pl.when(cid == 0)
    def _():
        pltpu.sync_copy(c_loc, cp_hbm.at[sid])


_edge_kernel = functools.partial(
    pl.kernel,
    out_type=[
        jax.ShapeDtypeStruct((NC, NS, RPS, FH), jnp.float32),
        jax.ShapeDtypeStruct((NS, NPAD), jnp.float32),
    ],
    mesh=plsc.VectorSubcoreMesh(core_axis_name="c", subcore_axis_name="s"),
    compiler_params=pltpu.CompilerParams(use_tc_tiling_on_sc=False,
                                         needs_layout_passes=False),
    scratch_types=[
        pltpu.VMEM((NCH, K), jnp.int32),
        pltpu.VMEM((NCH, K), jnp.int32),
        pltpu.VMEM((NCH, K), jnp.float32),
        pltpu.VMEM((K, FH), jnp.float32),
        pltpu.VMEM((K, FH), jnp.float32),
        pltpu.VMEM((K, FH), jnp.float32),
        pltpu.VMEM((K, FH), jnp.float32),
        pltpu.VMEM((NPAD,), jnp.float32),
        pltpu.VMEM_SHARED((N, FH), jnp.float32),
        pltpu.SemaphoreType.DMA,
        pltpu.SemaphoreType.DMA,
        pltpu.SemaphoreType.DMA,
        pltpu.SemaphoreType.DMA,
    ],
)(_edge_body)


def _finish_body(p_ref, cp_ref, w1_ref, w2_ref, o_ref):
    acc = jnp.concatenate([p_ref[0], p_ref[1]], axis=1)
    h1 = jnp.dot(acc, w1_ref[...], preferred_element_type=jnp.float32)
    r = jnp.maximum(h1, 0.0)
    c = jnp.sum(cp_ref[...], axis=0)[:N, None]
    s = jnp.sum(r * c, axis=0, keepdims=True)
    o_ref[...] = jnp.dot(s, w2_ref[...],
                         preferred_element_type=jnp.float32) * (1.0 / N)


def _finish(p, cp, w1, w2):
    return pl.pallas_call(
        _finish_body,
        out_shape=jax.ShapeDtypeStruct((1, NCLASS), jnp.float32),
    )(p, cp, w1, w2)


def kernel(x, edge_index, edge_weight, W1, W2):
    xs = x.reshape(N, NC, FH).transpose(1, 0, 2)
    src3d = edge_index[0].reshape(NS, NCH, K)
    dst3d = edge_index[1].reshape(NS, NCH, K)
    w3d = edge_weight.reshape(NS, NCH, K)
    accp, cp = _edge_kernel(xs, src3d, dst3d, w3d)
    accp = accp.reshape(NC, N, FH)
    return _finish(accp, cp, W1, W2)


# speedup vs baseline: 16.2758x; 1.0711x over previous
import functools

import numpy as np
import jax
import jax.numpy as jnp
from jax import lax
from jax.experimental import pallas as pl
from jax.experimental.pallas import tpu as pltpu
from jax.experimental.pallas import tpu_sc as plsc

N = 10000
E = 320000
F_IN = 128
HID = 128
FH = HID // 2
NCLASS = 16

NC = 2
NS = 16
EPS = E // NS
K = 80
NCH = EPS // K
RPS = N // NS
NPAD = 10240


def _edge_body(h_hbm, src_hbm, dst_hbm, w_hbm, accp_hbm, cp_hbm,
               srcb, dstb, wb, gbuf0, gbuf1, mbuf0, mbuf1, c_loc,
               acc_sh, gsem0, gsem1, ssem0, ssem1):
    cid = lax.axis_index("c")
    sid = lax.axis_index("s")
    gbuf = (gbuf0, gbuf1)
    mbuf = (mbuf0, mbuf1)
    gsem = (gsem0, gsem1)
    ssem = (ssem0, ssem1)

    zv = jnp.zeros((16,), jnp.float32)
    for p in range(2):
        for g in range(K):
            for k in range(FH // 16):
                mbuf[p][g, pl.ds(k * 16, 16)] = zv

    def zc(i, carry):
        c_loc[pl.ds(i * 16, 16)] = zv
        return carry

    lax.fori_loop(0, NPAD // 16, zc, 0)

    for t in range(8):
        base = sid * RPS + t * K
        sz = K if t < 7 else RPS - 7 * K
        pltpu.sync_copy(mbuf0.at[pl.ds(0, sz)], acc_sh.at[pl.ds(base, sz)])
    plsc.subcore_barrier()

    pltpu.sync_copy(src_hbm.at[sid], srcb)
    pltpu.sync_copy(dst_hbm.at[sid], dstb)
    pltpu.sync_copy(w_hbm.at[sid], wb)

    pltpu.async_copy(h_hbm.at[cid].at[srcb.at[0]], gbuf0, gsem0)
    for p in range(2):
        pltpu.async_copy(mbuf[p], acc_sh.at[dstb.at[p]], ssem[p], add=True)

    def step(t, carry):
        for p in range(2):
            j = 2 * t + p
            @pl.when(j + 1 < NCH)
            def _():
                pltpu.async_copy(h_hbm.at[cid].at[srcb.at[j + 1]],
                                 gbuf[1 - p], gsem[1 - p])

            pltpu.make_async_copy(h_hbm.at[cid].at[srcb.at[j]],
                                  gbuf[p], gsem[p]).wait()
            pltpu.make_async_copy(mbuf[p], acc_sh.at[dstb.at[j]],
                                  ssem[p]).wait()

            for g16 in range(K // 16):
                wv16 = wb[j, pl.ds(g16 * 16, 16)]
                idxv = srcb[j, pl.ds(g16 * 16, 16)]
                plsc.addupdate_scatter(c_loc, [idxv], wv16)
                for l in range(16):
                    g = g16 * 16 + l
                    wv = jnp.full((16,), wv16[l])
                    for k in range(FH // 32):
                        row = gbuf[p][g, pl.ds(k * 32, 32)]
                        a, b = plsc.unpack(
                            row, format=plsc.PackFormat.INTERLEAVED,
                            preferred_element_type=jnp.float32)
                        mbuf[p][g, pl.ds(k * 32, 16)] = a * wv
                        mbuf[p][g, pl.ds(k * 32 + 16, 16)] = b * wv
            pltpu.async_copy(mbuf[p], acc_sh.at[dstb.at[j]], ssem[p],
                             add=True)
        return carry

    lax.fori_loop(0, NCH // 2, step, 0)

    for p in range(2):
        pltpu.make_async_copy(mbuf[p], acc_sh.at[dstb.at[NCH - 2 + p]],
                              ssem[p]).wait()
    plsc.subcore_barrier()

    pltpu.sync_copy(acc_sh.at[pl.ds(sid * RPS, RPS)], accp_hbm.at[cid, sid])

    @pl.when(cid == 0)
    def _():
        pltpu.sync_copy(c_loc, cp_hbm.at[sid])


_edge_kernel = functools.partial(
    pl.kernel,
    out_type=[
        jax.ShapeDtypeStruct((NC, NS, RPS, FH), jnp.float32),
        jax.ShapeDtypeStruct((NS, NPAD), jnp.float32),
    ],
    mesh=plsc.VectorSubcoreMesh(core_axis_name="c", subcore_axis_name="s"),
    compiler_params=pltpu.CompilerParams(use_tc_tiling_on_sc=False,
                                         needs_layout_passes=False),
    scratch_types=[
        pltpu.VMEM((NCH, K), jnp.int32),
        pltpu.VMEM((NCH, K), jnp.int32),
        pltpu.VMEM((NCH, K), jnp.float32),
        pltpu.VMEM((K, FH), jnp.bfloat16),
        pltpu.VMEM((K, FH), jnp.bfloat16),
        pltpu.VMEM((K, FH), jnp.float32),
        pltpu.VMEM((K, FH), jnp.float32),
        pltpu.VMEM((NPAD,), jnp.float32),
        pltpu.VMEM_SHARED((N, FH), jnp.float32),
        pltpu.SemaphoreType.DMA,
        pltpu.SemaphoreType.DMA,
        pltpu.SemaphoreType.DMA,
        pltpu.SemaphoreType.DMA,
    ],
)(_edge_body)


def _finish_body(p_ref, cp_ref, w1_ref, w2_ref, o_ref):
    acc = jnp.concatenate([p_ref[0], p_ref[1]], axis=1)
    h1 = jnp.dot(acc, w1_ref[...], preferred_element_type=jnp.float32)
    r = jnp.maximum(h1, 0.0)
    c = jnp.sum(cp_ref[...], axis=0)[:N, None]
    s = jnp.sum(r * c, axis=0, keepdims=True)
    o_ref[...] = jnp.dot(s, w2_ref[...],
                         preferred_element_type=jnp.float32) * (1.0 / N)


def _finish(p, cp, w1, w2):
    return pl.pallas_call(
        _finish_body,
        out_shape=jax.ShapeDtypeStruct((1, NCLASS), jnp.float32),
    )(p, cp, w1, w2)


_PERM = np.concatenate(
    [np.concatenate([base + 2 * np.arange(16), base + 2 * np.arange(16) + 1])
     for base in range(0, HID, 32)])


def kernel(x, edge_index, edge_weight, W1, W2):
    xs = x.astype(jnp.bfloat16).reshape(N, NC, FH).transpose(1, 0, 2)
    src3d = edge_index[0].reshape(NS, NCH, K)
    dst3d = edge_index[1].reshape(NS, NCH, K)
    w3d = edge_weight.reshape(NS, NCH, K)
    accp, cp = _edge_kernel(xs, src3d, dst3d, w3d)
    accp = accp.reshape(NC, N, FH)
    return _finish(accp, cp, W1[_PERM, :], W2)


# speedup vs baseline: 17.4526x; 1.0723x over previous
import functools

import numpy as np
import jax
import jax.numpy as jnp
from jax import lax
from jax.experimental import pallas as pl
from jax.experimental.pallas import tpu as pltpu
from jax.experimental.pallas import tpu_sc as plsc

N = 10000
E = 320000
F_IN = 128
HID = 128
FH = HID // 2
NCLASS = 16

NC = 2
NS = 16
EPS = E // NS
K = 80
NCH = EPS // K
RPS = N // NS
NPAD = 10240


def _edge_body(h_hbm, src_hbm, dst_hbm, w_hbm, accp_hbm, cp_hbm,
               srcb, dstb, wb, gbuf0, gbuf1, mbuf0, mbuf1, c_loc,
               acc_sh, gsem0, gsem1, ssem0, ssem1):
    cid = lax.axis_index("c")
    sid = lax.axis_index("s")
    gbuf = (gbuf0, gbuf1)
    mbuf = (mbuf0, mbuf1)
    gsem = (gsem0, gsem1)
    ssem = (ssem0, ssem1)

    zv = jnp.zeros((16,), jnp.float32)
    zv32 = jnp.zeros((32,), jnp.bfloat16)
    for p in range(2):
        for g in range(K):
            for k in range(FH // 32):
                mbuf[p][g, pl.ds(k * 32, 32)] = zv32

    def zc(i, carry):
        c_loc[pl.ds(i * 16, 16)] = zv
        return carry

    lax.fori_loop(0, NPAD // 16, zc, 0)

    for t in range(8):
        base = sid * RPS + t * K
        sz = K if t < 7 else RPS - 7 * K
        pltpu.sync_copy(mbuf0.at[pl.ds(0, sz)], acc_sh.at[pl.ds(base, sz)])
    plsc.subcore_barrier()

    pltpu.sync_copy(src_hbm.at[sid], srcb)
    pltpu.sync_copy(dst_hbm.at[sid], dstb)
    pltpu.sync_copy(w_hbm.at[sid], wb)

    pltpu.async_copy(h_hbm.at[cid].at[srcb.at[0]], gbuf0, gsem0)
    for p in range(2):
        pltpu.async_copy(mbuf[p], acc_sh.at[dstb.at[p]], ssem[p], add=True)

    def step(t, carry):
        for p in range(2):
            j = 2 * t + p
            @pl.when(j + 1 < NCH)
            def _():
                pltpu.async_copy(h_hbm.at[cid].at[srcb.at[j + 1]],
                                 gbuf[1 - p], gsem[1 - p])

            pltpu.make_async_copy(h_hbm.at[cid].at[srcb.at[j]],
                                  gbuf[p], gsem[p]).wait()
            pltpu.make_async_copy(mbuf[p], acc_sh.at[dstb.at[j]],
                                  ssem[p]).wait()

            for g16 in range(K // 16):
                wv16 = wb[j, pl.ds(g16 * 16, 16)]
                idxv = srcb[j, pl.ds(g16 * 16, 16)]
                plsc.addupdate_scatter(c_loc, [idxv], wv16)
                for l in range(16):
                    g = g16 * 16 + l
                    wv = jnp.full((16,), wv16[l])
                    wpk = plsc.pack(wv, wv, format=plsc.PackFormat.INTERLEAVED)
                    for k in range(FH // 32):
                        mbuf[p][g, pl.ds(k * 32, 32)] = (
                            gbuf[p][g, pl.ds(k * 32, 32)] * wpk)
            pltpu.async_copy(mbuf[p], acc_sh.at[dstb.at[j]], ssem[p],
                             add=True)
        return carry

    lax.fori_loop(0, NCH // 2, step, 0)

    for p in range(2):
        pltpu.make_async_copy(mbuf[p], acc_sh.at[dstb.at[NCH - 2 + p]],
                              ssem[p]).wait()
    plsc.subcore_barrier()

    pltpu.sync_copy(acc_sh.at[pl.ds(sid * RPS, RPS)], accp_hbm.at[cid, sid])

    @pl.when(cid == 0)
    def _():
        pltpu.sync_copy(c_loc, cp_hbm.at[sid])


_edge_kernel = functools.partial(
    pl.kernel,
    out_type=[
        jax.ShapeDtypeStruct((NC, NS, RPS, FH), jnp.bfloat16),
        jax.ShapeDtypeStruct((NS, NPAD), jnp.float32),
    ],
    mesh=plsc.VectorSubcoreMesh(core_axis_name="c", subcore_axis_name="s"),
    compiler_params=pltpu.CompilerParams(use_tc_tiling_on_sc=False,
                                         needs_layout_passes=False),
    scratch_types=[
        pltpu.VMEM((NCH, K), jnp.int32),
        pltpu.VMEM((NCH, K), jnp.int32),
        pltpu.VMEM((NCH, K), jnp.float32),
        pltpu.VMEM((K, FH), jnp.bfloat16),
        pltpu.VMEM((K, FH), jnp.bfloat16),
        pltpu.VMEM((K, FH), jnp.bfloat16),
        pltpu.VMEM((K, FH), jnp.bfloat16),
        pltpu.VMEM((NPAD,), jnp.float32),
        pltpu.VMEM_SHARED((N, FH), jnp.bfloat16),
        pltpu.SemaphoreType.DMA,
        pltpu.SemaphoreType.DMA,
        pltpu.SemaphoreType.DMA,
        pltpu.SemaphoreType.DMA,
    ],
)(_edge_body)


def _finish_body(p_ref, cp_ref, w1_ref, w2_ref, o_ref):
    acc = jnp.concatenate([p_ref[0], p_ref[1]], axis=1)
    h1 = jnp.dot(acc.astype(jnp.float32), w1_ref[...],
                 preferred_element_type=jnp.float32)
    r = jnp.maximum(h1, 0.0)
    c = jnp.sum(cp_ref[...], axis=0)[:N, None]
    s = jnp.sum(r * c, axis=0, keepdims=True)
    o_ref[...] = jnp.dot(s, w2_ref[...],
                         preferred_element_type=jnp.float32) * (1.0 / N)


def _finish(p, cp, w1, w2):
    return pl.pallas_call(
        _finish_body,
        out_shape=jax.ShapeDtypeStruct((1, NCLASS), jnp.float32),
    )(p, cp, w1, w2)


def kernel(x, edge_index, edge_weight, W1, W2):
    xs = x.astype(jnp.bfloat16).reshape(N, NC, FH).transpose(1, 0, 2)
    src3d = edge_index[0].reshape(NS, NCH, K)
    dst3d = edge_index[1].reshape(NS, NCH, K)
    w3d = edge_weight.reshape(NS, NCH, K)
    accp, cp = _edge_kernel(xs, src3d, dst3d, w3d)
    accp = accp.reshape(NC, N, FH)
    return _finish(accp, cp, W1, W2)


# speedup vs baseline: 17.6542x; 1.0115x over previous
import functools

import numpy as np
import jax
import jax.numpy as jnp
from jax import lax
from jax.experimental import pallas as pl
from jax.experimental.pallas import tpu as pltpu
from jax.experimental.pallas import tpu_sc as plsc

N = 10000
E = 320000
F_IN = 128
HID = 128
FH = HID // 2
NCLASS = 16

NC = 2
NS = 16
EPS = E // NS
K = 80
NCH = EPS // K
RPS = N // NS
NPAD = 10240


def _split_body(x_ref, o_ref):
    o_ref[0] = x_ref[:, :FH].astype(jnp.bfloat16)
    o_ref[1] = x_ref[:, FH:].astype(jnp.bfloat16)


def _split_cast(x):
    bm = 400
    return pl.pallas_call(
        _split_body,
        grid=(N // bm,),
        in_specs=[pl.BlockSpec((bm, F_IN), lambda i: (i, 0))],
        out_specs=pl.BlockSpec((NC, bm, FH), lambda i: (0, i, 0)),
        out_shape=jax.ShapeDtypeStruct((NC, N, FH), jnp.bfloat16),
    )(x)


def _edge_body(h_hbm, ei_hbm, w_hbm, accp_hbm, cp_hbm,
               srcb, dstb, wb, gbuf0, gbuf1, mbuf0, mbuf1, c_loc,
               acc_sh, gsem0, gsem1, ssem0, ssem1):
    cid = lax.axis_index("c")
    sid = lax.axis_index("s")
    gbuf = (gbuf0, gbuf1)
    mbuf = (mbuf0, mbuf1)
    gsem = (gsem0, gsem1)
    ssem = (ssem0, ssem1)

    zv = jnp.zeros((16,), jnp.float32)
    zv32 = jnp.zeros((32,), jnp.bfloat16)
    for p in range(2):
        for g in range(K):
            for k in range(FH // 32):
                mbuf[p][g, pl.ds(k * 32, 32)] = zv32

    def zc(i, carry):
        c_loc[pl.ds(i * 16, 16)] = zv
        return carry

    lax.fori_loop(0, NPAD // 16, zc, 0)

    for t in range(8):
        base = sid * RPS + t * K
        sz = K if t < 7 else RPS - 7 * K
        pltpu.sync_copy(mbuf0.at[pl.ds(0, sz)], acc_sh.at[pl.ds(base, sz)])
    plsc.subcore_barrier()

    pltpu.sync_copy(ei_hbm.at[0].at[sid], srcb)
    pltpu.sync_copy(ei_hbm.at[1].at[sid], dstb)
    pltpu.sync_copy(w_hbm.at[sid], wb)

    pltpu.async_copy(h_hbm.at[cid].at[srcb.at[0]], gbuf0, gsem0)
    for p in range(2):
        pltpu.async_copy(mbuf[p], acc_sh.at[dstb.at[p]], ssem[p], add=True)

    def step(t, carry):
        for p in range(2):
            j = 2 * t + p
            @pl.when(j + 1 < NCH)
            def _():
                pltpu.async_copy(h_hbm.at[cid].at[srcb.at[j + 1]],
                                 gbuf[1 - p], gsem[1 - p])

            pltpu.make_async_copy(h_hbm.at[cid].at[srcb.at[j]],
                                  gbuf[p], gsem[p]).wait()
            pltpu.make_async_copy(mbuf[p], acc_sh.at[dstb.at[j]],
                                  ssem[p]).wait()

            for g16 in range(K // 16):
                wv16 = wb[j, pl.ds(g16 * 16, 16)]
                idxv = srcb[j, pl.ds(g16 * 16, 16)]
                plsc.addupdate_scatter(c_loc, [idxv], wv16)
                for l in range(16):
                    g = g16 * 16 + l
                    wv = jnp.full((16,), wv16[l])
                    wpk = plsc.pack(wv, wv, format=plsc.PackFormat.INTERLEAVED)
                    for k in range(FH // 32):
                        mbuf[p][g, pl.ds(k * 32, 32)] = (
                            gbuf[p][g, pl.ds(k * 32, 32)] * wpk)
            pltpu.async_copy(mbuf[p], acc_sh.at[dstb.at[j]], ssem[p],
                             add=True)
        return carry

    lax.fori_loop(0, NCH // 2, step, 0)

    for p in range(2):
        pltpu.make_async_copy(mbuf[p], acc_sh.at[dstb.at[NCH - 2 + p]],
                              ssem[p]).wait()
    plsc.subcore_barrier()

    pltpu.sync_copy(acc_sh.at[pl.ds(sid * RPS, RPS)], accp_hbm.at[cid, sid])

    @pl.when(cid == 0)
    def _():
        pltpu.sync_copy(c_loc, cp_hbm.at[sid])


_edge_kernel = functools.partial(
    pl.kernel,
    out_type=[
        jax.ShapeDtypeStruct((NC, NS, RPS, FH), jnp.bfloat16),
        jax.ShapeDtypeStruct((NS, NPAD), jnp.float32),
    ],
    mesh=plsc.VectorSubcoreMesh(core_axis_name="c", subcore_axis_name="s"),
    compiler_params=pltpu.CompilerParams(use_tc_tiling_on_sc=False,
                                         needs_layout_passes=False),
    scratch_types=[
        pltpu.VMEM((NCH, K), jnp.int32),
        pltpu.VMEM((NCH, K), jnp.int32),
        pltpu.VMEM((NCH, K), jnp.float32),
        pltpu.VMEM((K, FH), jnp.bfloat16),
        pltpu.VMEM((K, FH), jnp.bfloat16),
        pltpu.VMEM((K, FH), jnp.bfloat16),
        pltpu.VMEM((K, FH), jnp.bfloat16),
        pltpu.VMEM((NPAD,), jnp.float32),
        pltpu.VMEM_SHARED((N, FH), jnp.bfloat16),
        pltpu.SemaphoreType.DMA,
        pltpu.SemaphoreType.DMA,
        pltpu.SemaphoreType.DMA,
        pltpu.SemaphoreType.DMA,
    ],
)(_edge_body)


def _finish_body(p_ref, cp_ref, w1_ref, w2_ref, o_ref):
    acc = jnp.concatenate([p_ref[0], p_ref[1]], axis=1)
    h1 = jnp.dot(acc.astype(jnp.float32), w1_ref[...],
                 preferred_element_type=jnp.float32)
    r = jnp.maximum(h1, 0.0)
    c = jnp.sum(cp_ref[...], axis=0)[:N, None]
    s = jnp.sum(r * c, axis=0, keepdims=True)
    o_ref[...] = jnp.dot(s, w2_ref[...],
                         preferred_element_type=jnp.float32) * (1.0 / N)


def _finish(p, cp, w1, w2):
    return pl.pallas_call(
        _finish_body,
        out_shape=jax.ShapeDtypeStruct((1, NCLASS), jnp.float32),
    )(p, cp, w1, w2)


def kernel(x, edge_index, edge_weight, W1, W2):
    xs = _split_cast(x)
    ei4d = edge_index.reshape(2, NS, NCH, K)
    w3d = edge_weight.reshape(NS, NCH, K)
    accp, cp = _edge_kernel(xs, ei4d, w3d)
    accp = accp.reshape(NC, N, FH)
    return _finish(accp, cp, W1, W2)


# speedup vs baseline: 23.0175x; 1.3038x over previous
import functools

import numpy as np
import jax
import jax.numpy as jnp
from jax import lax
from jax.experimental import pallas as pl
from jax.experimental.pallas import tpu as pltpu
from jax.experimental.pallas import tpu_sc as plsc

N = 10000
E = 320000
F_IN = 128
HID = 128
FH = HID // 2
NCLASS = 16

NC = 2
NS = 16
EPS = E // NS
K = 80
NCH = EPS // K
RPS = N // NS
NPAD = 10240


def _split_body(x_ref, o_ref):
    o_ref[0] = x_ref[:, :FH].astype(jnp.bfloat16)
    o_ref[1] = x_ref[:, FH:].astype(jnp.bfloat16)


def _split_cast(x):
    bm = 400
    return pl.pallas_call(
        _split_body,
        grid=(N // bm,),
        in_specs=[pl.BlockSpec((bm, F_IN), lambda i: (i, 0))],
        out_specs=pl.BlockSpec((NC, bm, FH), lambda i: (0, i, 0)),
        out_shape=jax.ShapeDtypeStruct((NC, N, FH), jnp.bfloat16),
    )(x)


def _edge_body(h_hbm, ei_hbm, w_hbm, accp_hbm, cp_hbm,
               srcb, dstb, wb, gbuf0, gbuf1, mbuf0, mbuf1, c_loc,
               acc_sh, xs_sh, gsem0, gsem1, ssem0, ssem1):
    cid = lax.axis_index("c")
    sid = lax.axis_index("s")
    gbuf = (gbuf0, gbuf1)
    mbuf = (mbuf0, mbuf1)
    gsem = (gsem0, gsem1)
    ssem = (ssem0, ssem1)

    zv = jnp.zeros((16,), jnp.float32)
    zv32 = jnp.zeros((32,), jnp.bfloat16)
    for p in range(2):
        for g in range(K):
            for k in range(FH // 32):
                mbuf[p][g, pl.ds(k * 32, 32)] = zv32

    def zc(i, carry):
        c_loc[pl.ds(i * 16, 16)] = zv
        return carry

    lax.fori_loop(0, NPAD // 16, zc, 0)

    for t in range(8):
        base = sid * RPS + t * K
        sz = K if t < 7 else RPS - 7 * K
        pltpu.sync_copy(mbuf0.at[pl.ds(0, sz)], acc_sh.at[pl.ds(base, sz)])
    plsc.subcore_barrier()

    pltpu.sync_copy(ei_hbm.at[0].at[sid], srcb)
    pltpu.sync_copy(ei_hbm.at[1].at[sid], dstb)
    pltpu.sync_copy(w_hbm.at[sid], wb)
    pltpu.sync_copy(h_hbm.at[cid].at[pl.ds(sid * RPS, RPS)],
                    xs_sh.at[pl.ds(sid * RPS, RPS)])
    plsc.subcore_barrier()

    pltpu.async_copy(xs_sh.at[srcb.at[0]], gbuf0, gsem0)
    for p in range(2):
        pltpu.async_copy(mbuf[p], acc_sh.at[dstb.at[p]], ssem[p], add=True)

    def step(t, carry):
        for p in range(2):
            j = 2 * t + p
            @pl.when(j + 1 < NCH)
            def _():
                pltpu.async_copy(xs_sh.at[srcb.at[j + 1]],
                                 gbuf[1 - p], gsem[1 - p])

            pltpu.make_async_copy(xs_sh.at[srcb.at[j]],
                                  gbuf[p], gsem[p]).wait()
            pltpu.make_async_copy(mbuf[p], acc_sh.at[dstb.at[j]],
                                  ssem[p]).wait()

            for g16 in range(K // 16):
                wv16 = wb[j, pl.ds(g16 * 16, 16)]
                idxv = srcb[j, pl.ds(g16 * 16, 16)]
                plsc.addupdate_scatter(c_loc, [idxv], wv16)
                for l in range(16):
                    g = g16 * 16 + l
                    wv = jnp.full((16,), wv16[l])
                    wpk = plsc.pack(wv, wv, format=plsc.PackFormat.INTERLEAVED)
                    for k in range(FH // 32):
                        mbuf[p][g, pl.ds(k * 32, 32)] = (
                            gbuf[p][g, pl.ds(k * 32, 32)] * wpk)
            pltpu.async_copy(mbuf[p], acc_sh.at[dstb.at[j]], ssem[p],
                             add=True)
        return carry

    lax.fori_loop(0, NCH // 2, step, 0)

    for p in range(2):
        pltpu.make_async_copy(mbuf[p], acc_sh.at[dstb.at[NCH - 2 + p]],
                              ssem[p]).wait()
    plsc.subcore_barrier()

    pltpu.sync_copy(acc_sh.at[pl.ds(sid * RPS, RPS)], accp_hbm.at[cid, sid])

    @pl.when(cid == 0)
    def _():
        pltpu.sync_copy(c_loc, cp_hbm.at[sid])


_edge_kernel = functools.partial(
    pl.kernel,
    out_type=[
        jax.ShapeDtypeStruct((NC, NS, RPS, FH), jnp.bfloat16),
        jax.ShapeDtypeStruct((NS, NPAD), jnp.float32),
    ],
    mesh=plsc.VectorSubcoreMesh(core_axis_name="c", subcore_axis_name="s"),
    compiler_params=pltpu.CompilerParams(use_tc_tiling_on_sc=False,
                                         needs_layout_passes=False),
    scratch_types=[
        pltpu.VMEM((NCH, K), jnp.int32),
        pltpu.VMEM((NCH, K), jnp.int32),
        pltpu.VMEM((NCH, K), jnp.float32),
        pltpu.VMEM((K, FH), jnp.bfloat16),
        pltpu.VMEM((K, FH), jnp.bfloat16),
        pltpu.VMEM((K, FH), jnp.bfloat16),
        pltpu.VMEM((K, FH), jnp.bfloat16),
        pltpu.VMEM((NPAD,), jnp.float32),
        pltpu.VMEM_SHARED((N, FH), jnp.bfloat16),
        pltpu.VMEM_SHARED((N, FH), jnp.bfloat16),
        pltpu.SemaphoreType.DMA,
        pltpu.SemaphoreType.DMA,
        pltpu.SemaphoreType.DMA,
        pltpu.SemaphoreType.DMA,
    ],
)(_edge_body)


def _finish_body(p_ref, cp_ref, w1_ref, w2_ref, o_ref):
    acc = jnp.concatenate([p_ref[0], p_ref[1]], axis=1)
    h1 = jnp.dot(acc.astype(jnp.float32), w1_ref[...],
                 preferred_element_type=jnp.float32)
    r = jnp.maximum(h1, 0.0)
    c = jnp.sum(cp_ref[...], axis=0)[:N, None]
    s = jnp.sum(r * c, axis=0, keepdims=True)
    o_ref[...] = jnp.dot(s, w2_ref[...],
                         preferred_element_type=jnp.float32) * (1.0 / N)


def _finish(p, cp, w1, w2):
    return pl.pallas_call(
        _finish_body,
        out_shape=jax.ShapeDtypeStruct((1, NCLASS), jnp.float32),
    )(p, cp, w1, w2)


def kernel(x, edge_index, edge_weight, W1, W2):
    xs = _split_cast(x)
    ei4d = edge_index.reshape(2, NS, NCH, K)
    w3d = edge_weight.reshape(NS, NCH, K)
    accp, cp = _edge_kernel(xs, ei4d, w3d)
    accp = accp.reshape(NC, N, FH)
    return _finish(accp, cp, W1, W2)
